# baseline (speedup 1.0000x reference)
import functools

import jax
import jax.numpy as jnp
from jax import lax
from jax.experimental import pallas as pl
from jax.experimental.pallas import tpu as pltpu
from jax.experimental.pallas import tpu_sc as plsc

_TILE = 512
_CPAD = 1024
_BIG = 1e30
_D = 256
_IDX = 128


def _argmin_body(x_ref, c_ref, cn_ref, o_ref):
    x = x_ref[...]
    prod = jnp.dot(x, c_ref[...], preferred_element_type=jnp.float32)
    x2 = jnp.sum(x * x, axis=1, keepdims=True)
    dist = (x2 - 2.0 * prod) + cn_ref[...]
    m = jnp.min(dist, axis=1, keepdims=True)
    col = lax.broadcasted_iota(jnp.int32, dist.shape, 1)
    cand = jnp.where(dist <= m, col, _CPAD)
    o_ref[...] = jnp.min(cand, axis=1)[None, None, :]


def _cluster_ids(x, c_pad, cn_pad):
    n = x.shape[0]
    grid = n // _TILE
    return pl.pallas_call(
        _argmin_body,
        grid=(grid,),
        in_specs=[
            pl.BlockSpec((_TILE, x.shape[1]), lambda i: (i, 0)),
            pl.BlockSpec((x.shape[1], _CPAD), lambda i: (0, 0)),
            pl.BlockSpec((1, _CPAD), lambda i: (0, 0)),
        ],
        out_specs=pl.BlockSpec((1, 1, _TILE), lambda i: (i, 0, 0)),
        out_shape=jax.ShapeDtypeStruct((grid, 1, _TILE), jnp.int32),
    )(x, c_pad, cn_pad)


def _make_row_gather(n_rows):
    mesh = plsc.VectorSubcoreMesh(core_axis_name="c", subcore_axis_name="s")
    n_workers = mesh.num_cores * mesh.num_subcores
    per_w = n_rows // n_workers
    n_sub = per_w // _IDX

    @functools.partial(
        pl.kernel,
        mesh=mesh,
        out_type=jax.ShapeDtypeStruct((n_rows, _D), jnp.float32),
        scratch_types=[
            pltpu.VMEM((n_sub, _IDX), jnp.int32),
            pltpu.VMEM((_IDX, _D), jnp.float32),
            pltpu.VMEM((_IDX, _D), jnp.float32),
            pltpu.SemaphoreType.DMA,
            pltpu.SemaphoreType.DMA,
            pltpu.SemaphoreType.DMA,
            pltpu.SemaphoreType.DMA,
        ],
    )
    def gather_kernel(table_hbm, idx_hbm, out_hbm, idxv, b0, b1, g0, g1, w0, w1):
        wid = lax.axis_index("s") * mesh.num_cores + lax.axis_index("c")
        base = wid * per_w
        pltpu.sync_copy(idx_hbm.at[wid], idxv)
        bufs = (b0, b1)
        gsems = (g0, g1)
        wsems = (w0, w1)
        hg = [None] * n_sub
        hw = [None] * n_sub
        for k in range(min(2, n_sub)):
            hg[k] = pltpu.async_copy(table_hbm.at[idxv.at[k]], bufs[k], gsems[k])
        for k in range(n_sub):
            hg[k].wait()
            hw[k] = pltpu.async_copy(
                bufs[k % 2], out_hbm.at[pl.ds(base + k * _IDX, _IDX)], wsems[k % 2]
            )
            nk = k + 2
            if nk < n_sub:
                hw[k].wait()
                hg[nk] = pltpu.async_copy(
                    table_hbm.at[idxv.at[nk]], bufs[nk % 2], gsems[nk % 2]
                )
        for k in range(max(0, n_sub - 2), n_sub):
            hw[k].wait()

    return gather_kernel


def kernel(hs, C, Cnorm, emb_table):
    bs, seqlen, size = hs.shape
    n = bs * seqlen
    x = hs.reshape(n, size)
    pad = _CPAD - C.shape[1]
    c_pad = jnp.pad(C, ((0, 0), (0, pad)))
    cn_pad = jnp.pad(Cnorm, ((0, 0), (0, pad)), constant_values=_BIG)

    ids = _cluster_ids(x, c_pad, cn_pad)

    mesh_workers = 32
    idx3 = ids.reshape(mesh_workers, n // mesh_workers // _IDX, _IDX)
    gather = _make_row_gather(n)
    rows = gather(emb_table, idx3)
    return rows.reshape(bs, seqlen // 2, 2 * size)


# speedup vs baseline: 1.0740x; 1.0740x over previous
import functools

import jax
import jax.numpy as jnp
from jax import lax
from jax.experimental import pallas as pl
from jax.experimental.pallas import tpu as pltpu
from jax.experimental.pallas import tpu_sc as plsc

_TILE = 512
_CPAD = 1024
_BIG = 1e30
_D = 256
_IDX = 128


def _argmin_body(x_ref, c_ref, cn_ref, o_ref):
    x = x_ref[...]
    prod = jnp.dot(x, c_ref[...], preferred_element_type=jnp.float32)
    x2 = jnp.sum(x * x, axis=1, keepdims=True)
    dist = (x2 - 2.0 * prod) + cn_ref[...]
    m = jnp.min(dist, axis=1, keepdims=True)
    col = lax.broadcasted_iota(jnp.int32, dist.shape, 1)
    cand = jnp.where(dist <= m, col, _CPAD)
    o_ref[...] = jnp.min(cand, axis=1)[None, None, :]


def _cluster_ids(x, c_pad, cn_pad):
    n = x.shape[0]
    grid = n // _TILE
    return pl.pallas_call(
        _argmin_body,
        grid=(grid,),
        in_specs=[
            pl.BlockSpec((_TILE, x.shape[1]), lambda i: (i, 0)),
            pl.BlockSpec((x.shape[1], _CPAD), lambda i: (0, 0)),
            pl.BlockSpec((1, _CPAD), lambda i: (0, 0)),
        ],
        out_specs=pl.BlockSpec((1, 1, _TILE), lambda i: (i, 0, 0)),
        out_shape=jax.ShapeDtypeStruct((grid, 1, _TILE), jnp.int32),
    )(x, c_pad, cn_pad)


def _make_pair_gather(bs, positions, dd):
    mesh = plsc.VectorSubcoreMesh(core_axis_name="c", subcore_axis_name="s")
    n_workers = mesh.num_cores * mesh.num_subcores
    per_w = (bs * positions) // n_workers
    w_per_b = positions // per_w

    @functools.partial(
        pl.kernel,
        mesh=mesh,
        out_type=jax.ShapeDtypeStruct((bs, positions, dd), jnp.float32),
        scratch_types=[
            pltpu.VMEM((2, _IDX), jnp.int32),
            pltpu.VMEM((2, _IDX), jnp.int32),
            pltpu.VMEM((_IDX, _D), jnp.float32),
            pltpu.VMEM((_IDX, _D), jnp.float32),
            pltpu.VMEM((_IDX, _D), jnp.float32),
            pltpu.SemaphoreType.DMA,
            pltpu.SemaphoreType.DMA,
            pltpu.SemaphoreType.DMA,
            pltpu.SemaphoreType.DMA,
            pltpu.SemaphoreType.DMA,
            pltpu.SemaphoreType.DMA,
            pltpu.SemaphoreType.DMA,
            pltpu.SemaphoreType.DMA,
        ],
    )
    def gather_kernel(
        table_hbm, idxe_hbm, idxo_hbm, out_hbm,
        idxe, idxo, b0, b1, b2,
        g0, g1, g2, g3, w0, w1, w2, w3,
    ):
        wid = lax.axis_index("s") * mesh.num_cores + lax.axis_index("c")
        b = wid // w_per_b
        p0 = (wid % w_per_b) * per_w
        pltpu.sync_copy(idxe_hbm.at[wid], idxe)
        pltpu.sync_copy(idxo_hbm.at[wid], idxo)

        def oslice(k, half):
            return out_hbm.at[
                b, pl.ds(p0 + k * _IDX, _IDX), pl.ds(half * _D, _D)
            ]

        h0 = pltpu.async_copy(table_hbm.at[idxe.at[0]], b0, g0)
        h1 = pltpu.async_copy(table_hbm.at[idxo.at[0]], b1, g1)
        h0.wait()
        we0 = pltpu.async_copy(b0, oslice(0, 0), w0)
        h2 = pltpu.async_copy(table_hbm.at[idxe.at[1]], b2, g2)
        h1.wait()
        wo0 = pltpu.async_copy(b1, oslice(0, 1), w1)
        we0.wait()
        h3 = pltpu.async_copy(table_hbm.at[idxo.at[1]], b0, g3)
        h2.wait()
        we1 = pltpu.async_copy(b2, oslice(1, 0), w2)
        h3.wait()
        wo1 = pltpu.async_copy(b0, oslice(1, 1), w3)
        wo0.wait()
        we1.wait()
        wo1.wait()

    return gather_kernel


def kernel(hs, C, Cnorm, emb_table):
    bs, seqlen, size = hs.shape
    n = bs * seqlen
    x = hs.reshape(n, size)
    pad = _CPAD - C.shape[1]
    c_pad = jnp.pad(C, ((0, 0), (0, pad)))
    cn_pad = jnp.pad(Cnorm, ((0, 0), (0, pad)), constant_values=_BIG)

    ids = _cluster_ids(x, c_pad, cn_pad)

    n_workers = 32
    pairs = ids.reshape(n // 2, 2)
    idx_e = pairs[:, 0].reshape(n_workers, 2, _IDX)
    idx_o = pairs[:, 1].reshape(n_workers, 2, _IDX)
    gather = _make_pair_gather(bs, seqlen // 2, 2 * size)
    return gather(emb_table, idx_e, idx_o)


# speedup vs baseline: 1.6841x; 1.5681x over previous
import functools

import jax
import jax.numpy as jnp
from jax import lax
from jax.experimental import pallas as pl
from jax.experimental.pallas import tpu as pltpu
from jax.experimental.pallas import tpu_sc as plsc

_TILE = 512
_HALF = 256
_CPAD = 1024
_BIG = 1e30
_D = 256
_IDX = 128


def _argmin_body(x_ref, c_ref, cn_ref, o_ref):
    x = x_ref[...]
    prod = jnp.dot(x, c_ref[...], preferred_element_type=jnp.float32)
    x2 = jnp.sum(x * x, axis=1, keepdims=True)
    dist = (x2 - 2.0 * prod) + cn_ref[...]
    m = jnp.min(dist, axis=1, keepdims=True)
    col = lax.broadcasted_iota(jnp.int32, dist.shape, 1)
    cand = jnp.where(dist <= m, col, _CPAD)
    r = jnp.min(cand, axis=1).reshape(_HALF, 2)
    o_ref[0, 0, :] = r[:, 0]
    o_ref[0, 1, :] = r[:, 1]


def _cluster_ids(x, c_pad, cn_pad, tile0, tiles):
    return pl.pallas_call(
        _argmin_body,
        grid=(tiles,),
        in_specs=[
            pl.BlockSpec((_TILE, x.shape[1]), lambda i: (i + tile0, 0)),
            pl.BlockSpec((x.shape[1], _CPAD), lambda i: (0, 0)),
            pl.BlockSpec((1, _CPAD), lambda i: (0, 0)),
        ],
        out_specs=pl.BlockSpec((1, 2, _HALF), lambda i: (i, 0, 0)),
        out_shape=jax.ShapeDtypeStruct((tiles, 2, _HALF), jnp.int32),
    )(x, c_pad, cn_pad)


def _make_sc_gather(bs, positions, dd, batch0):
    mesh = plsc.VectorSubcoreMesh(core_axis_name="c", subcore_axis_name="s")
    tiles_per_b = positions // _HALF

    @functools.partial(
        pl.kernel,
        mesh=mesh,
        out_type=jax.ShapeDtypeStruct((bs, positions, dd), jnp.float32),
        scratch_types=[
            pltpu.VMEM((_IDX,), jnp.int32),
            pltpu.VMEM((_IDX,), jnp.int32),
            pltpu.VMEM((_IDX, _D), jnp.float32),
            pltpu.VMEM((_IDX, _D), jnp.float32),
            pltpu.SemaphoreType.DMA,
            pltpu.SemaphoreType.DMA,
            pltpu.SemaphoreType.DMA,
            pltpu.SemaphoreType.DMA,
        ],
    )
    def gather_kernel(table_hbm, ids_hbm, out_hbm, idxe, idxo, be, bo,
                      ge, go, we, wo):
        wid = lax.axis_index("s") * mesh.num_cores + lax.axis_index("c")
        t = wid // 2
        h = wid % 2
        b = batch0 + t // tiles_per_b
        p0 = (t % tiles_per_b) * _HALF + h * _IDX
        pltpu.sync_copy(ids_hbm.at[t, 0, pl.ds(h * _IDX, _IDX)], idxe)
        pltpu.sync_copy(ids_hbm.at[t, 1, pl.ds(h * _IDX, _IDX)], idxo)
        he = pltpu.async_copy(table_hbm.at[idxe], be, ge)
        ho = pltpu.async_copy(table_hbm.at[idxo], bo, go)
        he.wait()
        ce = pltpu.async_copy(be, out_hbm.at[b, pl.ds(p0, _IDX), pl.ds(0, _D)], we)
        ho.wait()
        co = pltpu.async_copy(bo, out_hbm.at[b, pl.ds(p0, _IDX), pl.ds(_D, _D)], wo)
        ce.wait()
        co.wait()

    return gather_kernel


def _onehot_body(ids_ref, emb_ref, _, o_ref):
    emb = emb_ref[...]
    iota = lax.broadcasted_iota(jnp.int32, (_HALF, _CPAD), 1)
    he = (ids_ref[0, 0, :][:, None] == iota).astype(jnp.float32)
    ho = (ids_ref[0, 1, :][:, None] == iota).astype(jnp.float32)
    o_ref[0, :, 0:_D] = jnp.dot(he, emb, preferred_element_type=jnp.float32)
    o_ref[0, :, _D:2 * _D] = jnp.dot(ho, emb, preferred_element_type=jnp.float32)


def _onehot_gather(ids, emb_pad, partial_out, tiles_per_b):
    tiles = ids.shape[0]
    bs, positions, dd = partial_out.shape
    return pl.pallas_call(
        _onehot_body,
        grid=(tiles,),
        in_specs=[
            pl.BlockSpec((1, 2, _HALF), lambda i: (i, 0, 0)),
            pl.BlockSpec((_CPAD, _D), lambda i: (0, 0)),
            pl.BlockSpec(memory_space=pl.ANY),
        ],
        out_specs=pl.BlockSpec(
            (1, _HALF, dd),
            lambda i, tb=tiles_per_b: (i // tb, i % tb, 0),
        ),
        out_shape=jax.ShapeDtypeStruct((bs, positions, dd), jnp.float32),
        input_output_aliases={2: 0},
    )(ids, emb_pad, partial_out)


def kernel(hs, C, Cnorm, emb_table):
    bs, seqlen, size = hs.shape
    n = bs * seqlen
    x = hs.reshape(n, size)
    pad = _CPAD - C.shape[1]
    c_pad = jnp.pad(C, ((0, 0), (0, pad)))
    cn_pad = jnp.pad(Cnorm, ((0, 0), (0, pad)), constant_values=_BIG)

    tiles = n // _TILE
    hi_tiles = tiles // 2
    positions = seqlen // 2
    tiles_per_b = positions // _HALF

    ids_hi = _cluster_ids(x, c_pad, cn_pad, hi_tiles, hi_tiles)
    sc_gather = _make_sc_gather(bs, positions, 2 * size, bs // 2)
    sc_out = sc_gather(emb_table, ids_hi)

    ids_lo = _cluster_ids(x, c_pad, cn_pad, 0, hi_tiles)
    emb_pad = jnp.pad(emb_table, ((0, _CPAD - emb_table.shape[0]), (0, 0)))
    return _onehot_gather(ids_lo, emb_pad, sc_out, tiles_per_b)


# speedup vs baseline: 1.8436x; 1.0947x over previous
import functools

import jax
import jax.numpy as jnp
from jax import lax
from jax.experimental import pallas as pl
from jax.experimental.pallas import tpu as pltpu
from jax.experimental.pallas import tpu_sc as plsc

_TILE = 512
_HALF = 256
_CPAD = 1024
_BIG = 1e30
_D = 256
_PIECE = 32
_SC_BATCHES = 3


def _argmin_flat_body(x_ref, c_ref, cn_ref, oe_ref, oo_ref):
    x = x_ref[...]
    prod = jnp.dot(x, c_ref[...], preferred_element_type=jnp.float32)
    x2 = jnp.sum(x * x, axis=1, keepdims=True)
    dist = (x2 - 2.0 * prod) + cn_ref[...]
    m = jnp.min(dist, axis=1, keepdims=True)
    col = lax.broadcasted_iota(jnp.int32, dist.shape, 1)
    cand = jnp.where(dist <= m, col, _CPAD)
    r = jnp.min(cand, axis=1).reshape(_HALF, 2)
    oe_ref[...] = r[:, 0]
    oo_ref[...] = r[:, 1]


def _argmin_tiled_body(x_ref, c_ref, cn_ref, o_ref):
    x = x_ref[...]
    prod = jnp.dot(x, c_ref[...], preferred_element_type=jnp.float32)
    x2 = jnp.sum(x * x, axis=1, keepdims=True)
    dist = (x2 - 2.0 * prod) + cn_ref[...]
    m = jnp.min(dist, axis=1, keepdims=True)
    col = lax.broadcasted_iota(jnp.int32, dist.shape, 1)
    cand = jnp.where(dist <= m, col, _CPAD)
    r = jnp.min(cand, axis=1).reshape(_HALF, 2)
    o_ref[0, 0, :] = r[:, 0]
    o_ref[0, 1, :] = r[:, 1]


def _cluster_ids_flat(x, c_pad, cn_pad, tile0, tiles):
    return pl.pallas_call(
        _argmin_flat_body,
        grid=(tiles,),
        in_specs=[
            pl.BlockSpec((_TILE, x.shape[1]), lambda i: (i + tile0, 0)),
            pl.BlockSpec((x.shape[1], _CPAD), lambda i: (0, 0)),
            pl.BlockSpec((1, _CPAD), lambda i: (0, 0)),
        ],
        out_specs=[
            pl.BlockSpec((_HALF,), lambda i: (i,)),
            pl.BlockSpec((_HALF,), lambda i: (i,)),
        ],
        out_shape=[
            jax.ShapeDtypeStruct((tiles * _HALF,), jnp.int32),
            jax.ShapeDtypeStruct((tiles * _HALF,), jnp.int32),
        ],
    )(x, c_pad, cn_pad)


def _cluster_ids_tiled(x, c_pad, cn_pad, tile0, tiles):
    return pl.pallas_call(
        _argmin_tiled_body,
        grid=(tiles,),
        in_specs=[
            pl.BlockSpec((_TILE, x.shape[1]), lambda i: (i + tile0, 0)),
            pl.BlockSpec((x.shape[1], _CPAD), lambda i: (0, 0)),
            pl.BlockSpec((1, _CPAD), lambda i: (0, 0)),
        ],
        out_specs=pl.BlockSpec((1, 2, _HALF), lambda i: (i, 0, 0)),
        out_shape=jax.ShapeDtypeStruct((tiles, 2, _HALF), jnp.int32),
    )(x, c_pad, cn_pad)


def _make_sc_gather(bs, positions, dd, batch0):
    mesh = plsc.VectorSubcoreMesh(core_axis_name="c", subcore_axis_name="s")
    n_workers = mesh.num_cores * mesh.num_subcores
    n_pos = (bs - batch0) * positions
    per_w = n_pos // n_workers
    n_pieces = per_w // _PIECE
    pos_base = batch0 * positions

    @functools.partial(
        pl.kernel,
        mesh=mesh,
        out_type=jax.ShapeDtypeStruct((bs, positions, dd), jnp.float32),
        scratch_types=[
            pltpu.VMEM((per_w,), jnp.int32),
            pltpu.VMEM((per_w,), jnp.int32),
            pltpu.VMEM((per_w, _D), jnp.float32),
            pltpu.VMEM((per_w, _D), jnp.float32),
            pltpu.SemaphoreType.DMA,
            pltpu.SemaphoreType.DMA,
            pltpu.SemaphoreType.DMA,
            pltpu.SemaphoreType.DMA,
        ],
    )
    def gather_kernel(table_hbm, ide_hbm, ido_hbm, out_hbm, idxe, idxo,
                      be, bo, ge, go, we, wo):
        wid = lax.axis_index("s") * mesh.num_cores + lax.axis_index("c")
        w0 = wid * per_w
        pltpu.sync_copy(ide_hbm.at[pl.ds(w0, per_w)], idxe)
        pltpu.sync_copy(ido_hbm.at[pl.ds(w0, per_w)], idxo)
        he = pltpu.async_copy(table_hbm.at[idxe], be, ge)
        ho = pltpu.async_copy(table_hbm.at[idxo], bo, go)

        def pieces(buf, half, sem):
            handles = []
            for j in range(n_pieces):
                pos = pos_base + w0 + j * _PIECE
                b = pos // positions
                p = pos % positions
                dst = out_hbm.at[b, pl.ds(p, _PIECE), pl.ds(half * _D, _D)]
                handles.append(
                    pltpu.async_copy(buf.at[pl.ds(j * _PIECE, _PIECE)],
                                     dst, sem)
                )
            return handles

        he.wait()
        ce = pieces(be, 0, we)
        ho.wait()
        co = pieces(bo, 1, wo)
        for h in ce + co:
            h.wait()

    return gather_kernel


def _onehot_body(ids_ref, emb_ref, _, o_ref):
    emb = emb_ref[...]
    iota = lax.broadcasted_iota(jnp.int32, (_HALF, _CPAD), 1)
    he = (ids_ref[0, 0, :][:, None] == iota).astype(jnp.float32)
    ho = (ids_ref[0, 1, :][:, None] == iota).astype(jnp.float32)
    o_ref[0, :, 0:_D] = jnp.dot(he, emb, preferred_element_type=jnp.float32)
    o_ref[0, :, _D:2 * _D] = jnp.dot(ho, emb, preferred_element_type=jnp.float32)


def _onehot_gather(ids, emb_pad, partial_out, tiles_per_b):
    tiles = ids.shape[0]
    bs, positions, dd = partial_out.shape
    return pl.pallas_call(
        _onehot_body,
        grid=(tiles,),
        in_specs=[
            pl.BlockSpec((1, 2, _HALF), lambda i: (i, 0, 0)),
            pl.BlockSpec((_CPAD, _D), lambda i: (0, 0)),
            pl.BlockSpec(memory_space=pl.ANY),
        ],
        out_specs=pl.BlockSpec(
            (1, _HALF, dd),
            lambda i, tb=tiles_per_b: (i // tb, i % tb, 0),
        ),
        out_shape=jax.ShapeDtypeStruct((bs, positions, dd), jnp.float32),
        input_output_aliases={2: 0},
    )(ids, emb_pad, partial_out)


def kernel(hs, C, Cnorm, emb_table):
    bs, seqlen, size = hs.shape
    n = bs * seqlen
    x = hs.reshape(n, size)
    pad = _CPAD - C.shape[1]
    c_pad = jnp.pad(C, ((0, 0), (0, pad)))
    cn_pad = jnp.pad(Cnorm, ((0, 0), (0, pad)), constant_values=_BIG)

    positions = seqlen // 2
    tiles_per_b = positions // _HALF
    lo_batches = bs - _SC_BATCHES
    lo_tiles = lo_batches * tiles_per_b
    hi_tiles = _SC_BATCHES * tiles_per_b

    ids_hi_e, ids_hi_o = _cluster_ids_flat(x, c_pad, cn_pad, lo_tiles,
                                           hi_tiles)
    sc_gather = _make_sc_gather(bs, positions, 2 * size, lo_batches)
    sc_out = sc_gather(emb_table, ids_hi_e, ids_hi_o)

    ids_lo = _cluster_ids_tiled(x, c_pad, cn_pad, 0, lo_tiles)
    emb_pad = jnp.pad(emb_table, ((0, _CPAD - emb_table.shape[0]), (0, 0)))
    return _onehot_gather(ids_lo, emb_pad, sc_out, tiles_per_b)
